# baseline (speedup 1.0000x reference)
import functools

import jax
import jax.numpy as jnp
from jax import lax
from jax.experimental import pallas as pl
from jax.experimental.pallas import tpu as pltpu
from jax.experimental.pallas import tpu_sc as plsc

NE = 16
K = 2
D = 1024
DFF = 1024
N = 4096
CAP = 640
NCOPY = N * K
ROWS = NE * CAP
ROWS_PAD = ROWS + 8

NW = 32
TOK_PER_W = N // NW
DISPATCH_CHUNK = 32
COMBINE_CHUNK = 16

_HIGH = lax.Precision.HIGHEST


def _routing_body(f_ref, w_ref, counts_ref, scat_ref, comb_ref, weff_ref):
    F = f_ref[...]
    W = w_ref[...]
    r0 = lax.broadcasted_iota(jnp.int32, (128, 128), 0)
    c0 = lax.broadcasted_iota(jnp.int32, (128, 128), 1)
    T = (r0 < c0).astype(jnp.float32)
    Ls = (r0[:64, :64] > c0[:64, :64]).astype(jnp.float32)

    ones = jnp.ones((128, 1), jnp.float32)
    ohs, inrows, cnts = [], [], []
    for e in range(NE):
        oe = (F == e).astype(jnp.float32)
        ohs.append(oe)
        inrows.append(lax.dot(oe, T, precision=_HIGH))
        cnts.append(lax.dot(oe, ones, precision=_HIGH))
    cnt = jnp.concatenate(cnts, axis=1)
    roff = lax.dot(Ls, cnt, precision=_HIGH)

    rank = jnp.zeros((64, 128), jnp.float32)
    for e in range(NE):
        rank = rank + ohs[e] * (inrows[e] + roff[:, e:e + 1])
    rank = rank.astype(jnp.int32)

    counts = jnp.sum(cnt, axis=0, keepdims=True)
    counts_ref[...] = jnp.concatenate(
        [counts, jnp.zeros((1, 112), jnp.float32)], axis=1).astype(jnp.int32)

    keep = rank < CAP
    slot = F * CAP + rank
    scat_ref[...] = jnp.where(keep, slot, ROWS)
    slot00 = jnp.broadcast_to(slot[0:1, 0:1], (64, 128))
    comb_ref[...] = jnp.where(keep, slot, slot00)
    weff_ref[...] = jnp.where(keep, W, 0.0)


def _routing(flat_idx, flat_w):
    return pl.pallas_call(
        _routing_body,
        out_shape=(
            jax.ShapeDtypeStruct((1, 128), jnp.int32),
            jax.ShapeDtypeStruct((64, 128), jnp.int32),
            jax.ShapeDtypeStruct((64, 128), jnp.int32),
            jax.ShapeDtypeStruct((64, 128), jnp.float32),
        ),
    )(flat_idx.reshape(64, 128), flat_w.reshape(64, 128))


def _dispatch_body(x_hbm, sa_hbm, sb_hbm, buf_hbm,
                   xv0, xv1, ia0, ia1, ib0, ib1, sx0, sx1, sca, scb):
    xv_ = (xv0, xv1)
    ia_ = (ia0, ia1)
    ib_ = (ib0, ib1)
    sx_ = (sx0, sx1)
    wid = lax.axis_index("s") * 2 + lax.axis_index("c")
    base = wid * TOK_PER_W
    nchunks = TOK_PER_W // DISPATCH_CHUNK
    loads = [None, None]
    scats = [None, None]

    def fire_load(k):
        s = k % 2
        tb = pl.multiple_of(base + k * DISPATCH_CHUNK, DISPATCH_CHUNK)
        pltpu.sync_copy(sa_hbm.at[pl.ds(tb, DISPATCH_CHUNK)], ia_[s])
        pltpu.sync_copy(sb_hbm.at[pl.ds(tb, DISPATCH_CHUNK)], ib_[s])
        loads[s] = pltpu.async_copy(x_hbm.at[pl.ds(tb, DISPATCH_CHUNK)],
                                    xv_[s], sx_[s])

    fire_load(0)
    fire_load(1)
    for k in range(nchunks):
        s = k % 2
        loads[s].wait()
        scats[s] = (pltpu.async_copy(xv_[s], buf_hbm.at[ia_[s]], sca),
                    pltpu.async_copy(xv_[s], buf_hbm.at[ib_[s]], scb))
        if k + 2 < nchunks:
            nxt = (k + 2) % 2
            for c in scats[nxt]:
                c.wait()
            scats[nxt] = None
            fire_load(k + 2)
    for pair in scats:
        if pair is not None:
            for c in pair:
                c.wait()


def _dispatch(x, sa, sb):
    return pl.kernel(
        _dispatch_body,
        out_type=jax.ShapeDtypeStruct((ROWS_PAD, D), jnp.float32),
        mesh=plsc.VectorSubcoreMesh(core_axis_name="c", subcore_axis_name="s", num_cores=2, num_subcores=16),
        scratch_types=[
            pltpu.VMEM((DISPATCH_CHUNK, D), jnp.float32),
            pltpu.VMEM((DISPATCH_CHUNK, D), jnp.float32),
            pltpu.VMEM((DISPATCH_CHUNK,), jnp.int32),
            pltpu.VMEM((DISPATCH_CHUNK,), jnp.int32),
            pltpu.VMEM((DISPATCH_CHUNK,), jnp.int32),
            pltpu.VMEM((DISPATCH_CHUNK,), jnp.int32),
            pltpu.SemaphoreType.DMA,
            pltpu.SemaphoreType.DMA,
            pltpu.SemaphoreType.DMA,
            pltpu.SemaphoreType.DMA,
        ],
    )(x, sa, sb)


def _mlp_body(x_ref, w1_ref, w3_ref, w2_ref, y_ref):
    f = pl.program_id(1)
    xb = x_ref[...].astype(jnp.bfloat16)
    w1 = w1_ref[0].astype(jnp.bfloat16)
    w3 = w3_ref[0].astype(jnp.bfloat16)
    w2 = w2_ref[0].astype(jnp.bfloat16)
    a = lax.dot(xb, w1, preferred_element_type=jnp.float32)
    b = lax.dot(xb, w3, preferred_element_type=jnp.float32)
    h = (a * jax.nn.sigmoid(a) * b).astype(jnp.bfloat16)
    yp = lax.dot(h, w2, preferred_element_type=jnp.float32)

    @pl.when(f == 0)
    def _():
        y_ref[...] = yp

    @pl.when(f != 0)
    def _():
        y_ref[...] += yp


def _mlp(buf, w1, w3, w2):
    return pl.pallas_call(
        _mlp_body,
        grid=(NE, 2),
        in_specs=[
            pl.BlockSpec((CAP, D), lambda e, f: (e, 0)),
            pl.BlockSpec((1, D, DFF // 2), lambda e, f: (e, 0, f)),
            pl.BlockSpec((1, D, DFF // 2), lambda e, f: (e, 0, f)),
            pl.BlockSpec((1, DFF // 2, D), lambda e, f: (e, f, 0)),
        ],
        out_specs=pl.BlockSpec((CAP, D), lambda e, f: (e, 0)),
        out_shape=jax.ShapeDtypeStruct((ROWS, D), jnp.float32),
    )(buf, w1, w3, w2)


def _combine_body(y_hbm, ca_hbm, cb_hbm, wa_hbm, wb_hbm, out_hbm,
                  ya0, ya1, yb0, yb1, ov, ia0, ia1, ib0, ib1, wav, wbv,
                  sa0, sa1, sb0, sb1):
    ya_ = (ya0, ya1)
    yb_ = (yb0, yb1)
    ia_ = (ia0, ia1)
    ib_ = (ib0, ib1)
    sa_ = (sa0, sa1)
    sb_ = (sb0, sb1)
    wid = lax.axis_index("s") * 2 + lax.axis_index("c")
    base = pl.multiple_of(wid * TOK_PER_W, TOK_PER_W)
    pltpu.sync_copy(wa_hbm.at[pl.ds(base * 16, TOK_PER_W * 16)], wav)
    pltpu.sync_copy(wb_hbm.at[pl.ds(base * 16, TOK_PER_W * 16)], wbv)
    nchunks = TOK_PER_W // COMBINE_CHUNK

    def fire(k):
        s = k % 2
        tb = pl.multiple_of(base + k * COMBINE_CHUNK, COMBINE_CHUNK)
        pltpu.sync_copy(ca_hbm.at[pl.ds(tb, COMBINE_CHUNK)], ia_[s])
        pltpu.sync_copy(cb_hbm.at[pl.ds(tb, COMBINE_CHUNK)], ib_[s])
        pltpu.async_copy(y_hbm.at[ia_[s]], ya_[s], sa_[s])
        pltpu.async_copy(y_hbm.at[ib_[s]], yb_[s], sb_[s])

    fire(0)
    for k in range(nchunks):
        s = k % 2
        if k + 1 < nchunks:
            fire(k + 1)
        tb = pl.multiple_of(base + k * COMBINE_CHUNK, COMBINE_CHUNK)
        pltpu.make_async_copy(y_hbm.at[ia_[s]], ya_[s], sa_[s]).wait()
        pltpu.make_async_copy(y_hbm.at[ib_[s]], yb_[s], sb_[s]).wait()
        ya, yb = ya_[s], yb_[s]

        def tok(t, _):
            ti = k * COMBINE_CHUNK + t
            ws = pl.ds(pl.multiple_of(ti * 16, 16), 16)
            wa = wav[ws]
            wb = wbv[ws]
            for j in range(D // 16):
                cs = pl.ds(j * 16, 16)
                ov[t, cs] = ya[t, cs] * wa + yb[t, cs] * wb
            return 0

        lax.fori_loop(0, COMBINE_CHUNK, tok, 0)
        pltpu.sync_copy(ov, out_hbm.at[pl.ds(tb, COMBINE_CHUNK)])


def _combine(y, ca, cb, wa, wb):
    return pl.kernel(
        _combine_body,
        out_type=jax.ShapeDtypeStruct((N, D), jnp.float32),
        mesh=plsc.VectorSubcoreMesh(core_axis_name="c", subcore_axis_name="s", num_cores=2, num_subcores=16),
        scratch_types=[
            pltpu.VMEM((COMBINE_CHUNK, D), jnp.float32),
            pltpu.VMEM((COMBINE_CHUNK, D), jnp.float32),
            pltpu.VMEM((COMBINE_CHUNK, D), jnp.float32),
            pltpu.VMEM((COMBINE_CHUNK, D), jnp.float32),
            pltpu.VMEM((COMBINE_CHUNK, D), jnp.float32),
            pltpu.VMEM((COMBINE_CHUNK,), jnp.int32),
            pltpu.VMEM((COMBINE_CHUNK,), jnp.int32),
            pltpu.VMEM((COMBINE_CHUNK,), jnp.int32),
            pltpu.VMEM((COMBINE_CHUNK,), jnp.int32),
            pltpu.VMEM((TOK_PER_W * 16,), jnp.float32),
            pltpu.VMEM((TOK_PER_W * 16,), jnp.float32),
            pltpu.SemaphoreType.DMA,
            pltpu.SemaphoreType.DMA,
            pltpu.SemaphoreType.DMA,
            pltpu.SemaphoreType.DMA,
        ],
    )(y, ca, cb, wa, wb)


def kernel(x, expert_weights, expert_indices, w1, w3, w2):
    flat_idx = expert_indices.reshape(-1).astype(jnp.int32)
    flat_w = expert_weights.reshape(-1)
    counts_p, scat, comb, weff = _routing(flat_idx, flat_w)
    counts = counts_p[0, :NE]

    scat = scat.reshape(N, K)
    comb = comb.reshape(N, K)
    weff = weff.reshape(N, K)

    buf = _dispatch(x, scat[:, 0], scat[:, 1])
    y = _mlp(buf, w1, w3, w2)
    wa16 = jnp.broadcast_to(weff[:, 0:1], (N, 16)).reshape(N * 16)
    wb16 = jnp.broadcast_to(weff[:, 1:2], (N, 16)).reshape(N * 16)
    out = _combine(y, comb[:, 0], comb[:, 1], wa16, wb16)
    return out, counts


# speedup vs baseline: 1.0422x; 1.0422x over previous
import jax
import jax.numpy as jnp
from jax import lax
from jax.experimental import pallas as pl
from jax.experimental.pallas import tpu as pltpu
from jax.experimental.pallas import tpu_sc as plsc

NE = 16
K = 2
D = 1024
DFF = 1024
N = 4096
CAP = 640
NCOPY = N * K
ROWS = NE * CAP
ROWS_PAD = ROWS + 8

NW = 32
TOK_PER_W = N // NW
DISPATCH_CHUNK = 32
COMBINE_CHUNK = 16

_HIGH = lax.Precision.HIGHEST


def _routing_body(f_ref, w_ref, counts_ref, scat_ref, comb_ref, weff_ref):
    F = f_ref[...]
    W = w_ref[...]
    r0 = lax.broadcasted_iota(jnp.int32, (128, 128), 0)
    c0 = lax.broadcasted_iota(jnp.int32, (128, 128), 1)
    T = (r0 < c0).astype(jnp.float32)
    Ls = (r0[:64, :64] > c0[:64, :64]).astype(jnp.float32)

    ones = jnp.ones((128, 1), jnp.float32)
    ohs, inrows, cnts = [], [], []
    for e in range(NE):
        oe = (F == e).astype(jnp.float32)
        ohs.append(oe)
        inrows.append(lax.dot(oe, T, precision=_HIGH))
        cnts.append(lax.dot(oe, ones, precision=_HIGH))
    cnt = jnp.concatenate(cnts, axis=1)
    roff = lax.dot(Ls, cnt, precision=_HIGH)

    rank = jnp.zeros((64, 128), jnp.float32)
    for e in range(NE):
        rank = rank + ohs[e] * (inrows[e] + roff[:, e:e + 1])
    rank = rank.astype(jnp.int32)

    counts = jnp.sum(cnt, axis=0, keepdims=True)
    counts_ref[...] = jnp.concatenate(
        [counts, jnp.zeros((1, 112), jnp.float32)], axis=1).astype(jnp.int32)

    keep = rank < CAP
    slot = F * CAP + rank
    scat_ref[...] = jnp.where(keep, slot, ROWS)
    slot00 = jnp.broadcast_to(slot[0:1, 0:1], (64, 128))
    comb_ref[...] = jnp.where(keep, slot, slot00)
    weff_ref[...] = jnp.where(keep, W, 0.0)


def _routing(flat_idx, flat_w):
    return pl.pallas_call(
        _routing_body,
        out_shape=(
            jax.ShapeDtypeStruct((1, 128), jnp.int32),
            jax.ShapeDtypeStruct((64, 128), jnp.int32),
            jax.ShapeDtypeStruct((64, 128), jnp.int32),
            jax.ShapeDtypeStruct((64, 128), jnp.float32),
        ),
    )(flat_idx.reshape(64, 128), flat_w.reshape(64, 128))


def _dispatch_body(x_hbm, sa_hbm, sb_hbm, buf_hbm,
                   xv0, xv1, ia0, ia1, ib0, ib1, sx0, sx1, sca, scb):
    xv_ = (xv0, xv1)
    ia_ = (ia0, ia1)
    ib_ = (ib0, ib1)
    sx_ = (sx0, sx1)
    wid = lax.axis_index("s") * 2 + lax.axis_index("c")
    base = wid * TOK_PER_W
    nchunks = TOK_PER_W // DISPATCH_CHUNK
    loads = [None, None]
    scats = [None, None]

    def fire_load(k):
        s = k % 2
        tb = pl.multiple_of(base + k * DISPATCH_CHUNK, DISPATCH_CHUNK)
        pltpu.sync_copy(sa_hbm.at[pl.ds(tb, DISPATCH_CHUNK)], ia_[s])
        pltpu.sync_copy(sb_hbm.at[pl.ds(tb, DISPATCH_CHUNK)], ib_[s])
        loads[s] = pltpu.async_copy(x_hbm.at[pl.ds(tb, DISPATCH_CHUNK)],
                                    xv_[s], sx_[s])

    fire_load(0)
    fire_load(1)
    for k in range(nchunks):
        s = k % 2
        loads[s].wait()
        scats[s] = (pltpu.async_copy(xv_[s], buf_hbm.at[ia_[s]], sca),
                    pltpu.async_copy(xv_[s], buf_hbm.at[ib_[s]], scb))
        if k + 2 < nchunks:
            nxt = (k + 2) % 2
            for c in scats[nxt]:
                c.wait()
            scats[nxt] = None
            fire_load(k + 2)
    for pair in scats:
        if pair is not None:
            for c in pair:
                c.wait()


def _dispatch(x, sa, sb):
    return pl.kernel(
        _dispatch_body,
        out_type=jax.ShapeDtypeStruct((ROWS_PAD, D), jnp.float32),
        mesh=plsc.VectorSubcoreMesh(core_axis_name="c", subcore_axis_name="s", num_cores=2, num_subcores=16),
        scratch_types=[
            pltpu.VMEM((DISPATCH_CHUNK, D), jnp.float32),
            pltpu.VMEM((DISPATCH_CHUNK, D), jnp.float32),
            pltpu.VMEM((DISPATCH_CHUNK,), jnp.int32),
            pltpu.VMEM((DISPATCH_CHUNK,), jnp.int32),
            pltpu.VMEM((DISPATCH_CHUNK,), jnp.int32),
            pltpu.VMEM((DISPATCH_CHUNK,), jnp.int32),
            pltpu.SemaphoreType.DMA,
            pltpu.SemaphoreType.DMA,
            pltpu.SemaphoreType.DMA,
            pltpu.SemaphoreType.DMA,
        ],
    )(x, sa, sb)


def _mlp_body(x_ref, w1_ref, w3_ref, w2_ref, y_ref):
    xb = x_ref[...].astype(jnp.bfloat16)
    w1 = w1_ref[0].astype(jnp.bfloat16)
    w3 = w3_ref[0].astype(jnp.bfloat16)
    w2 = w2_ref[0].astype(jnp.bfloat16)
    a = lax.dot(xb, w1, preferred_element_type=jnp.float32)
    b = lax.dot(xb, w3, preferred_element_type=jnp.float32)
    h = (a * jax.nn.sigmoid(a) * b).astype(jnp.bfloat16)
    y_ref[...] = lax.dot(h, w2, preferred_element_type=jnp.float32)


def _mlp(buf, w1, w3, w2):
    return pl.pallas_call(
        _mlp_body,
        grid=(NE,),
        in_specs=[
            pl.BlockSpec((CAP, D), lambda e: (e, 0)),
            pl.BlockSpec((1, D, DFF), lambda e: (e, 0, 0)),
            pl.BlockSpec((1, D, DFF), lambda e: (e, 0, 0)),
            pl.BlockSpec((1, DFF, D), lambda e: (e, 0, 0)),
        ],
        out_specs=pl.BlockSpec((CAP, D), lambda e: (e, 0)),
        out_shape=jax.ShapeDtypeStruct((ROWS, D), jnp.float32),
    )(buf, w1, w3, w2)


def _combine_body(y_hbm, ca_hbm, cb_hbm, wa_hbm, wb_hbm, out_hbm,
                  ya0, ya1, yb0, yb1, ov, ia0, ia1, ib0, ib1, wav, wbv,
                  sa0, sa1, sb0, sb1):
    ya_ = (ya0, ya1)
    yb_ = (yb0, yb1)
    ia_ = (ia0, ia1)
    ib_ = (ib0, ib1)
    sa_ = (sa0, sa1)
    sb_ = (sb0, sb1)
    wid = lax.axis_index("s") * 2 + lax.axis_index("c")
    base = pl.multiple_of(wid * TOK_PER_W, TOK_PER_W)
    pltpu.sync_copy(wa_hbm.at[pl.ds(base * 16, TOK_PER_W * 16)], wav)
    pltpu.sync_copy(wb_hbm.at[pl.ds(base * 16, TOK_PER_W * 16)], wbv)
    nchunks = TOK_PER_W // COMBINE_CHUNK

    def fire(k):
        s = k % 2
        tb = pl.multiple_of(base + k * COMBINE_CHUNK, COMBINE_CHUNK)
        pltpu.sync_copy(ca_hbm.at[pl.ds(tb, COMBINE_CHUNK)], ia_[s])
        pltpu.sync_copy(cb_hbm.at[pl.ds(tb, COMBINE_CHUNK)], ib_[s])
        pltpu.async_copy(y_hbm.at[ia_[s]], ya_[s], sa_[s])
        pltpu.async_copy(y_hbm.at[ib_[s]], yb_[s], sb_[s])

    fire(0)
    for k in range(nchunks):
        s = k % 2
        if k + 1 < nchunks:
            fire(k + 1)
        tb = pl.multiple_of(base + k * COMBINE_CHUNK, COMBINE_CHUNK)
        pltpu.make_async_copy(y_hbm.at[ia_[s]], ya_[s], sa_[s]).wait()
        pltpu.make_async_copy(y_hbm.at[ib_[s]], yb_[s], sb_[s]).wait()
        ya, yb = ya_[s], yb_[s]

        def tok(t, _):
            ti = k * COMBINE_CHUNK + t
            ws = pl.ds(pl.multiple_of(ti * 16, 16), 16)
            wa = wav[ws]
            wb = wbv[ws]
            for j in range(D // 16):
                cs = pl.ds(j * 16, 16)
                ov[t, cs] = ya[t, cs] * wa + yb[t, cs] * wb
            return 0

        lax.fori_loop(0, COMBINE_CHUNK, tok, 0)
        pltpu.sync_copy(ov, out_hbm.at[pl.ds(tb, COMBINE_CHUNK)])


def _combine(y, ca, cb, wa, wb):
    return pl.kernel(
        _combine_body,
        out_type=jax.ShapeDtypeStruct((N, D), jnp.float32),
        mesh=plsc.VectorSubcoreMesh(core_axis_name="c", subcore_axis_name="s", num_cores=2, num_subcores=16),
        scratch_types=[
            pltpu.VMEM((COMBINE_CHUNK, D), jnp.float32),
            pltpu.VMEM((COMBINE_CHUNK, D), jnp.float32),
            pltpu.VMEM((COMBINE_CHUNK, D), jnp.float32),
            pltpu.VMEM((COMBINE_CHUNK, D), jnp.float32),
            pltpu.VMEM((COMBINE_CHUNK, D), jnp.float32),
            pltpu.VMEM((COMBINE_CHUNK,), jnp.int32),
            pltpu.VMEM((COMBINE_CHUNK,), jnp.int32),
            pltpu.VMEM((COMBINE_CHUNK,), jnp.int32),
            pltpu.VMEM((COMBINE_CHUNK,), jnp.int32),
            pltpu.VMEM((TOK_PER_W * 16,), jnp.float32),
            pltpu.VMEM((TOK_PER_W * 16,), jnp.float32),
            pltpu.SemaphoreType.DMA,
            pltpu.SemaphoreType.DMA,
            pltpu.SemaphoreType.DMA,
            pltpu.SemaphoreType.DMA,
        ],
    )(y, ca, cb, wa, wb)


def kernel(x, expert_weights, expert_indices, w1, w3, w2):
    flat_idx = expert_indices.reshape(-1).astype(jnp.int32)
    flat_w = expert_weights.reshape(-1)
    counts_p, scat, comb, weff = _routing(flat_idx, flat_w)
    counts = counts_p[0, :NE]

    scat = scat.reshape(N, K)
    comb = comb.reshape(N, K)
    weff = weff.reshape(N, K)

    buf = _dispatch(x, scat[:, 0], scat[:, 1])
    y = _mlp(buf, w1, w3, w2)
    wa16 = jnp.broadcast_to(weff[:, 0:1], (N, 16)).reshape(N * 16)
    wb16 = jnp.broadcast_to(weff[:, 1:2], (N, 16)).reshape(N * 16)
    out = _combine(y, comb[:, 0], comb[:, 1], wa16, wb16)
    return out, counts


# speedup vs baseline: 1.0844x; 1.0405x over previous
import jax
import jax.numpy as jnp
from jax import lax
from jax.experimental import pallas as pl
from jax.experimental.pallas import tpu as pltpu
from jax.experimental.pallas import tpu_sc as plsc

NE = 16
K = 2
D = 1024
DFF = 1024
N = 4096
CAP = 640
NCOPY = N * K
ROWS = NE * CAP
ROWS_PAD = ROWS + 8

NW = 32
TOK_PER_W = N // NW
DISPATCH_CHUNK = 32
COMBINE_CHUNK = 16

_HIGH = lax.Precision.HIGHEST


def _routing_body(f_ref, w_ref, counts_ref, scat_ref, comb_ref, weff_ref):
    F = f_ref[...]
    W = w_ref[...]
    r0 = lax.broadcasted_iota(jnp.int32, (128, 128), 0)
    c0 = lax.broadcasted_iota(jnp.int32, (128, 128), 1)
    T = (r0 < c0).astype(jnp.float32)
    Ls = (r0[:64, :64] > c0[:64, :64]).astype(jnp.float32)

    ones = jnp.ones((128, 1), jnp.float32)
    ohs, inrows, cnts = [], [], []
    for e in range(NE):
        oe = (F == e).astype(jnp.float32)
        ohs.append(oe)
        inrows.append(lax.dot(oe, T, precision=_HIGH))
        cnts.append(lax.dot(oe, ones, precision=_HIGH))
    cnt = jnp.concatenate(cnts, axis=1)
    roff = lax.dot(Ls, cnt, precision=_HIGH)

    rank = jnp.zeros((64, 128), jnp.float32)
    for e in range(NE):
        rank = rank + ohs[e] * (inrows[e] + roff[:, e:e + 1])
    rank = rank.astype(jnp.int32)

    counts = jnp.sum(cnt, axis=0, keepdims=True)
    counts_ref[...] = jnp.concatenate(
        [counts, jnp.zeros((1, 112), jnp.float32)], axis=1).astype(jnp.int32)

    keep = rank < CAP
    slot = F * CAP + rank
    scat_ref[...] = jnp.where(keep, slot, ROWS)
    slot00 = jnp.broadcast_to(slot[0:1, 0:1], (64, 128))
    comb_ref[...] = jnp.where(keep, slot, slot00)
    weff_ref[...] = jnp.where(keep, W, 0.0)


def _routing(flat_idx, flat_w):
    return pl.pallas_call(
        _routing_body,
        out_shape=(
            jax.ShapeDtypeStruct((1, 128), jnp.int32),
            jax.ShapeDtypeStruct((64, 128), jnp.int32),
            jax.ShapeDtypeStruct((64, 128), jnp.int32),
            jax.ShapeDtypeStruct((64, 128), jnp.float32),
        ),
    )(flat_idx.reshape(64, 128), flat_w.reshape(64, 128))


def _dispatch_body(x_hbm, sa_hbm, sb_hbm, buf_hbm,
                   xv0, xv1, ia0, ia1, ib0, ib1, sx0, sx1, sca, scb):
    xv_ = (xv0, xv1)
    ia_ = (ia0, ia1)
    ib_ = (ib0, ib1)
    sx_ = (sx0, sx1)
    wid = lax.axis_index("s") * 2 + lax.axis_index("c")
    base = wid * TOK_PER_W
    nchunks = TOK_PER_W // DISPATCH_CHUNK
    loads = [None, None]
    scats = [None, None]

    def fire_load(k):
        s = k % 2
        tb = pl.multiple_of(base + k * DISPATCH_CHUNK, DISPATCH_CHUNK)
        pltpu.sync_copy(sa_hbm.at[pl.ds(tb, DISPATCH_CHUNK)], ia_[s])
        pltpu.sync_copy(sb_hbm.at[pl.ds(tb, DISPATCH_CHUNK)], ib_[s])
        loads[s] = pltpu.async_copy(x_hbm.at[pl.ds(tb, DISPATCH_CHUNK)],
                                    xv_[s], sx_[s])

    fire_load(0)
    fire_load(1)
    for k in range(nchunks):
        s = k % 2
        loads[s].wait()
        scats[s] = (pltpu.async_copy(xv_[s], buf_hbm.at[ia_[s]], sca),
                    pltpu.async_copy(xv_[s], buf_hbm.at[ib_[s]], scb))
        if k + 2 < nchunks:
            nxt = (k + 2) % 2
            for c in scats[nxt]:
                c.wait()
            scats[nxt] = None
            fire_load(k + 2)
    for pair in scats:
        if pair is not None:
            for c in pair:
                c.wait()


def _dispatch(x, sa, sb):
    return pl.kernel(
        _dispatch_body,
        out_type=jax.ShapeDtypeStruct((ROWS_PAD, D), jnp.float32),
        mesh=plsc.VectorSubcoreMesh(core_axis_name="c", subcore_axis_name="s", num_cores=2, num_subcores=16),
        scratch_types=[
            pltpu.VMEM((DISPATCH_CHUNK, D), jnp.float32),
            pltpu.VMEM((DISPATCH_CHUNK, D), jnp.float32),
            pltpu.VMEM((DISPATCH_CHUNK,), jnp.int32),
            pltpu.VMEM((DISPATCH_CHUNK,), jnp.int32),
            pltpu.VMEM((DISPATCH_CHUNK,), jnp.int32),
            pltpu.VMEM((DISPATCH_CHUNK,), jnp.int32),
            pltpu.SemaphoreType.DMA,
            pltpu.SemaphoreType.DMA,
            pltpu.SemaphoreType.DMA,
            pltpu.SemaphoreType.DMA,
        ],
    )(x, sa, sb)


def _mlp_body(x_ref, w1_ref, w3_ref, w2_ref, y_ref):
    xb = x_ref[...].astype(jnp.bfloat16)
    w1 = w1_ref[0].astype(jnp.bfloat16)
    w3 = w3_ref[0].astype(jnp.bfloat16)
    w2 = w2_ref[0].astype(jnp.bfloat16)
    a = lax.dot(xb, w1, preferred_element_type=jnp.float32)
    b = lax.dot(xb, w3, preferred_element_type=jnp.float32)
    h = (a * jax.nn.sigmoid(a) * b).astype(jnp.bfloat16)
    y_ref[...] = lax.dot(h, w2, preferred_element_type=jnp.float32)


def _mlp(buf, w1, w3, w2):
    return pl.pallas_call(
        _mlp_body,
        grid=(NE,),
        in_specs=[
            pl.BlockSpec((CAP, D), lambda e: (e, 0)),
            pl.BlockSpec((1, D, DFF), lambda e: (e, 0, 0)),
            pl.BlockSpec((1, D, DFF), lambda e: (e, 0, 0)),
            pl.BlockSpec((1, DFF, D), lambda e: (e, 0, 0)),
        ],
        out_specs=pl.BlockSpec((CAP, D), lambda e: (e, 0)),
        out_shape=jax.ShapeDtypeStruct((ROWS, D), jnp.float32),
    )(buf, w1, w3, w2)


def _combine_body(y_hbm, ca_hbm, cb_hbm, wa_hbm, wb_hbm, out_hbm,
                  ya0, ya1, yb0, yb1, ov0, ov1, ia0, ia1, ib0, ib1, wav, wbv,
                  sa0, sa1, sb0, sb1, so0, so1):
    ya_ = (ya0, ya1)
    yb_ = (yb0, yb1)
    ov_ = (ov0, ov1)
    ia_ = (ia0, ia1)
    ib_ = (ib0, ib1)
    sa_ = (sa0, sa1)
    sb_ = (sb0, sb1)
    so_ = (so0, so1)
    outs = [None, None]
    wid = lax.axis_index("s") * 2 + lax.axis_index("c")
    base = pl.multiple_of(wid * TOK_PER_W, TOK_PER_W)
    pltpu.sync_copy(wa_hbm.at[pl.ds(base * 16, TOK_PER_W * 16)], wav)
    pltpu.sync_copy(wb_hbm.at[pl.ds(base * 16, TOK_PER_W * 16)], wbv)
    nchunks = TOK_PER_W // COMBINE_CHUNK

    def fire(k):
        s = k % 2
        tb = pl.multiple_of(base + k * COMBINE_CHUNK, COMBINE_CHUNK)
        pltpu.sync_copy(ca_hbm.at[pl.ds(tb, COMBINE_CHUNK)], ia_[s])
        pltpu.sync_copy(cb_hbm.at[pl.ds(tb, COMBINE_CHUNK)], ib_[s])
        pltpu.async_copy(y_hbm.at[ia_[s]], ya_[s], sa_[s])
        pltpu.async_copy(y_hbm.at[ib_[s]], yb_[s], sb_[s])

    fire(0)
    for k in range(nchunks):
        s = k % 2
        if k + 1 < nchunks:
            fire(k + 1)
        tb = pl.multiple_of(base + k * COMBINE_CHUNK, COMBINE_CHUNK)
        pltpu.make_async_copy(y_hbm.at[ia_[s]], ya_[s], sa_[s]).wait()
        pltpu.make_async_copy(y_hbm.at[ib_[s]], yb_[s], sb_[s]).wait()
        if outs[s] is not None:
            outs[s].wait()
            outs[s] = None
        ya, yb, ov = ya_[s], yb_[s], ov_[s]

        def tok(t, _):
            ti = k * COMBINE_CHUNK + t
            ws = pl.ds(pl.multiple_of(ti * 16, 16), 16)
            wa = wav[ws]
            wb = wbv[ws]
            for j in range(D // 16):
                cs = pl.ds(j * 16, 16)
                ov[t, cs] = ya[t, cs] * wa + yb[t, cs] * wb
            return 0

        lax.fori_loop(0, COMBINE_CHUNK, tok, 0)
        outs[s] = pltpu.async_copy(ov, out_hbm.at[pl.ds(tb, COMBINE_CHUNK)],
                                   so_[s])
    for d in outs:
        if d is not None:
            d.wait()


def _combine(y, ca, cb, wa, wb):
    return pl.kernel(
        _combine_body,
        out_type=jax.ShapeDtypeStruct((N, D), jnp.float32),
        mesh=plsc.VectorSubcoreMesh(core_axis_name="c", subcore_axis_name="s", num_cores=2, num_subcores=16),
        scratch_types=[
            pltpu.VMEM((COMBINE_CHUNK, D), jnp.float32),
            pltpu.VMEM((COMBINE_CHUNK, D), jnp.float32),
            pltpu.VMEM((COMBINE_CHUNK, D), jnp.float32),
            pltpu.VMEM((COMBINE_CHUNK, D), jnp.float32),
            pltpu.VMEM((COMBINE_CHUNK, D), jnp.float32),
            pltpu.VMEM((COMBINE_CHUNK, D), jnp.float32),
            pltpu.VMEM((COMBINE_CHUNK,), jnp.int32),
            pltpu.VMEM((COMBINE_CHUNK,), jnp.int32),
            pltpu.VMEM((COMBINE_CHUNK,), jnp.int32),
            pltpu.VMEM((COMBINE_CHUNK,), jnp.int32),
            pltpu.VMEM((TOK_PER_W * 16,), jnp.float32),
            pltpu.VMEM((TOK_PER_W * 16,), jnp.float32),
            pltpu.SemaphoreType.DMA,
            pltpu.SemaphoreType.DMA,
            pltpu.SemaphoreType.DMA,
            pltpu.SemaphoreType.DMA,
            pltpu.SemaphoreType.DMA,
            pltpu.SemaphoreType.DMA,
        ],
    )(y, ca, cb, wa, wb)


def kernel(x, expert_weights, expert_indices, w1, w3, w2):
    flat_idx = expert_indices.reshape(-1).astype(jnp.int32)
    flat_w = expert_weights.reshape(-1)
    counts_p, scat, comb, weff = _routing(flat_idx, flat_w)
    counts = counts_p[0, :NE]

    scat = scat.reshape(N, K)
    comb = comb.reshape(N, K)
    weff = weff.reshape(N, K)

    buf = _dispatch(x, scat[:, 0], scat[:, 1])
    y = _mlp(buf, w1, w3, w2)
    wa16 = jnp.broadcast_to(weff[:, 0:1], (N, 16)).reshape(N * 16)
    wb16 = jnp.broadcast_to(weff[:, 1:2], (N, 16)).reshape(N * 16)
    out = _combine(y, comb[:, 0], comb[:, 1], wa16, wb16)
    return out, counts


# speedup vs baseline: 1.0958x; 1.0105x over previous
import jax
import jax.numpy as jnp
from jax import lax
from jax.experimental import pallas as pl
from jax.experimental.pallas import tpu as pltpu
from jax.experimental.pallas import tpu_sc as plsc

NE = 16
K = 2
D = 1024
DFF = 1024
N = 4096
CAP = 640
NCOPY = N * K
ROWS = NE * CAP
ROWS_PAD = ROWS + 8

NW = 32
TOK_PER_W = N // NW
DISPATCH_CHUNK = 32
COMBINE_CHUNK = 16

_HIGH = lax.Precision.HIGHEST


def _routing_body(f_ref, w_ref, counts_ref, scat_ref, comb_ref, weff_ref):
    F = f_ref[...]
    W = w_ref[...]
    r0 = lax.broadcasted_iota(jnp.int32, (128, 128), 0)
    c0 = lax.broadcasted_iota(jnp.int32, (128, 128), 1)
    T = (r0 < c0).astype(jnp.float32)
    Ls = (r0[:64, :64] > c0[:64, :64]).astype(jnp.float32)

    ones = jnp.ones((128, 1), jnp.float32)
    ohs, inrows, cnts = [], [], []
    for e in range(NE):
        oe = (F == e).astype(jnp.float32)
        ohs.append(oe)
        inrows.append(lax.dot(oe, T, precision=_HIGH))
        cnts.append(lax.dot(oe, ones, precision=_HIGH))
    cnt = jnp.concatenate(cnts, axis=1)
    roff = lax.dot(Ls, cnt, precision=_HIGH)

    rank = jnp.zeros((64, 128), jnp.float32)
    for e in range(NE):
        rank = rank + ohs[e] * (inrows[e] + roff[:, e:e + 1])
    rank = rank.astype(jnp.int32)

    counts = jnp.sum(cnt, axis=0, keepdims=True)
    counts_ref[...] = jnp.concatenate(
        [counts, jnp.zeros((1, 112), jnp.float32)], axis=1).astype(jnp.int32)

    keep = rank < CAP
    slot = F * CAP + rank
    scat_ref[...] = jnp.where(keep, slot, ROWS)
    slot00 = jnp.broadcast_to(slot[0:1, 0:1], (64, 128))
    comb_ref[...] = jnp.where(keep, slot, slot00)
    weff_ref[...] = jnp.where(keep, W, 0.0)


def _routing(flat_idx, flat_w):
    return pl.pallas_call(
        _routing_body,
        out_shape=(
            jax.ShapeDtypeStruct((1, 128), jnp.int32),
            jax.ShapeDtypeStruct((64, 128), jnp.int32),
            jax.ShapeDtypeStruct((64, 128), jnp.int32),
            jax.ShapeDtypeStruct((64, 128), jnp.float32),
        ),
    )(flat_idx.reshape(64, 128), flat_w.reshape(64, 128))


def _dispatch_body(x_hbm, sa_hbm, sb_hbm, buf_hbm,
                   xv0, xv1, ia0, ia1, ib0, ib1, sx0, sx1, si0, si1,
                   sca, scb):
    xv_ = (xv0, xv1)
    ia_ = (ia0, ia1)
    ib_ = (ib0, ib1)
    sx_ = (sx0, sx1)
    si_ = (si0, si1)
    wid = lax.axis_index("s") * 2 + lax.axis_index("c")
    base = wid * TOK_PER_W
    nchunks = TOK_PER_W // DISPATCH_CHUNK
    loads = [None, None]
    idxls = [None, None]
    scats = [None, None]

    def fire_load(k):
        s = k % 2
        tb = pl.multiple_of(base + k * DISPATCH_CHUNK, DISPATCH_CHUNK)
        idxls[s] = (
            pltpu.async_copy(sa_hbm.at[pl.ds(tb, DISPATCH_CHUNK)], ia_[s],
                             si_[s]),
            pltpu.async_copy(sb_hbm.at[pl.ds(tb, DISPATCH_CHUNK)], ib_[s],
                             si_[s]),
        )
        loads[s] = pltpu.async_copy(x_hbm.at[pl.ds(tb, DISPATCH_CHUNK)],
                                    xv_[s], sx_[s])

    fire_load(0)
    fire_load(1)
    for k in range(nchunks):
        s = k % 2
        loads[s].wait()
        for d in idxls[s]:
            d.wait()
        scats[s] = (pltpu.async_copy(xv_[s], buf_hbm.at[ia_[s]], sca),
                    pltpu.async_copy(xv_[s], buf_hbm.at[ib_[s]], scb))
        if k + 2 < nchunks:
            nxt = (k + 2) % 2
            for c in scats[nxt]:
                c.wait()
            scats[nxt] = None
            fire_load(k + 2)
    for pair in scats:
        if pair is not None:
            for c in pair:
                c.wait()


def _dispatch(x, sa, sb):
    return pl.kernel(
        _dispatch_body,
        out_type=jax.ShapeDtypeStruct((ROWS_PAD, D), jnp.float32),
        mesh=plsc.VectorSubcoreMesh(core_axis_name="c", subcore_axis_name="s", num_cores=2, num_subcores=16),
        scratch_types=[
            pltpu.VMEM((DISPATCH_CHUNK, D), jnp.float32),
            pltpu.VMEM((DISPATCH_CHUNK, D), jnp.float32),
            pltpu.VMEM((DISPATCH_CHUNK,), jnp.int32),
            pltpu.VMEM((DISPATCH_CHUNK,), jnp.int32),
            pltpu.VMEM((DISPATCH_CHUNK,), jnp.int32),
            pltpu.VMEM((DISPATCH_CHUNK,), jnp.int32),
            pltpu.SemaphoreType.DMA,
            pltpu.SemaphoreType.DMA,
            pltpu.SemaphoreType.DMA,
            pltpu.SemaphoreType.DMA,
            pltpu.SemaphoreType.DMA,
            pltpu.SemaphoreType.DMA,
        ],
    )(x, sa, sb)


def _mlp_body(x_ref, w1_ref, w3_ref, w2_ref, y_ref):
    xb = x_ref[...].astype(jnp.bfloat16)
    w1 = w1_ref[0].astype(jnp.bfloat16)
    w3 = w3_ref[0].astype(jnp.bfloat16)
    w2 = w2_ref[0].astype(jnp.bfloat16)
    a = lax.dot(xb, w1, preferred_element_type=jnp.float32)
    b = lax.dot(xb, w3, preferred_element_type=jnp.float32)
    h = (a * jax.nn.sigmoid(a) * b).astype(jnp.bfloat16)
    y_ref[...] = lax.dot(h, w2, preferred_element_type=jnp.float32)


def _mlp(buf, w1, w3, w2):
    return pl.pallas_call(
        _mlp_body,
        grid=(NE,),
        in_specs=[
            pl.BlockSpec((CAP, D), lambda e: (e, 0)),
            pl.BlockSpec((1, D, DFF), lambda e: (e, 0, 0)),
            pl.BlockSpec((1, D, DFF), lambda e: (e, 0, 0)),
            pl.BlockSpec((1, DFF, D), lambda e: (e, 0, 0)),
        ],
        out_specs=pl.BlockSpec((CAP, D), lambda e: (e, 0)),
        out_shape=jax.ShapeDtypeStruct((ROWS, D), jnp.float32),
    )(buf, w1, w3, w2)


def _combine_body(y_hbm, ca_hbm, cb_hbm, wa_hbm, wb_hbm, out_hbm,
                  ya0, ya1, yb0, yb1, ov0, ov1, ia0, ia1, ib0, ib1, wav, wbv,
                  sa0, sa1, sb0, sb1, so0, so1):
    ya_ = (ya0, ya1)
    yb_ = (yb0, yb1)
    ov_ = (ov0, ov1)
    ia_ = (ia0, ia1)
    ib_ = (ib0, ib1)
    sa_ = (sa0, sa1)
    sb_ = (sb0, sb1)
    so_ = (so0, so1)
    outs = [None, None]
    wid = lax.axis_index("s") * 2 + lax.axis_index("c")
    base = pl.multiple_of(wid * TOK_PER_W, TOK_PER_W)
    pltpu.sync_copy(wa_hbm.at[pl.ds(base * 16, TOK_PER_W * 16)], wav)
    pltpu.sync_copy(wb_hbm.at[pl.ds(base * 16, TOK_PER_W * 16)], wbv)
    nchunks = TOK_PER_W // COMBINE_CHUNK

    def fire(k):
        s = k % 2
        tb = pl.multiple_of(base + k * COMBINE_CHUNK, COMBINE_CHUNK)
        pltpu.sync_copy(ca_hbm.at[pl.ds(tb, COMBINE_CHUNK)], ia_[s])
        pltpu.sync_copy(cb_hbm.at[pl.ds(tb, COMBINE_CHUNK)], ib_[s])
        pltpu.async_copy(y_hbm.at[ia_[s]], ya_[s], sa_[s])
        pltpu.async_copy(y_hbm.at[ib_[s]], yb_[s], sb_[s])

    fire(0)
    for k in range(nchunks):
        s = k % 2
        if k + 1 < nchunks:
            fire(k + 1)
        tb = pl.multiple_of(base + k * COMBINE_CHUNK, COMBINE_CHUNK)
        pltpu.make_async_copy(y_hbm.at[ia_[s]], ya_[s], sa_[s]).wait()
        pltpu.make_async_copy(y_hbm.at[ib_[s]], yb_[s], sb_[s]).wait()
        if outs[s] is not None:
            outs[s].wait()
            outs[s] = None
        ya, yb, ov = ya_[s], yb_[s], ov_[s]

        def tok(t, _):
            ti = k * COMBINE_CHUNK + t
            ws = pl.ds(pl.multiple_of(ti * 16, 16), 16)
            wa = wav[ws]
            wb = wbv[ws]
            for j in range(D // 16):
                cs = pl.ds(j * 16, 16)
                ov[t, cs] = ya[t, cs] * wa + yb[t, cs] * wb
            return 0

        lax.fori_loop(0, COMBINE_CHUNK, tok, 0)
        outs[s] = pltpu.async_copy(ov, out_hbm.at[pl.ds(tb, COMBINE_CHUNK)],
                                   so_[s])
    for d in outs:
        if d is not None:
            d.wait()


def _combine(y, ca, cb, wa, wb):
    return pl.kernel(
        _combine_body,
        out_type=jax.ShapeDtypeStruct((N, D), jnp.float32),
        mesh=plsc.VectorSubcoreMesh(core_axis_name="c", subcore_axis_name="s", num_cores=2, num_subcores=16),
        scratch_types=[
            pltpu.VMEM((COMBINE_CHUNK, D), jnp.float32),
            pltpu.VMEM((COMBINE_CHUNK, D), jnp.float32),
            pltpu.VMEM((COMBINE_CHUNK, D), jnp.float32),
            pltpu.VMEM((COMBINE_CHUNK, D), jnp.float32),
            pltpu.VMEM((COMBINE_CHUNK, D), jnp.float32),
            pltpu.VMEM((COMBINE_CHUNK, D), jnp.float32),
            pltpu.VMEM((COMBINE_CHUNK,), jnp.int32),
            pltpu.VMEM((COMBINE_CHUNK,), jnp.int32),
            pltpu.VMEM((COMBINE_CHUNK,), jnp.int32),
            pltpu.VMEM((COMBINE_CHUNK,), jnp.int32),
            pltpu.VMEM((TOK_PER_W * 16,), jnp.float32),
            pltpu.VMEM((TOK_PER_W * 16,), jnp.float32),
            pltpu.SemaphoreType.DMA,
            pltpu.SemaphoreType.DMA,
            pltpu.SemaphoreType.DMA,
            pltpu.SemaphoreType.DMA,
            pltpu.SemaphoreType.DMA,
            pltpu.SemaphoreType.DMA,
        ],
    )(y, ca, cb, wa, wb)


def kernel(x, expert_weights, expert_indices, w1, w3, w2):
    flat_idx = expert_indices.reshape(-1).astype(jnp.int32)
    flat_w = expert_weights.reshape(-1)
    counts_p, scat, comb, weff = _routing(flat_idx, flat_w)
    counts = counts_p[0, :NE]

    scat = scat.reshape(N, K)
    comb = comb.reshape(N, K)
    weff = weff.reshape(N, K)

    buf = _dispatch(x, scat[:, 0], scat[:, 1])
    y = _mlp(buf, w1, w3, w2)
    wa16 = jnp.broadcast_to(weff[:, 0:1], (N, 16)).reshape(N * 16)
    wb16 = jnp.broadcast_to(weff[:, 1:2], (N, 16)).reshape(N * 16)
    out = _combine(y, comb[:, 0], comb[:, 1], wa16, wb16)
    return out, counts
